# Initial kernel scaffold; baseline (speedup 1.0000x reference)
import functools

import jax
import jax.numpy as jnp
from jax import lax
from jax.experimental import pallas as pl
from jax.experimental.pallas import tpu as pltpu
from jax.experimental.pallas import tpu_sc as plsc

N = 10000
E = 320000
D = 128
NUM_EMB = 100000

NC = 2
NS = 16
NW = NC * NS

G = 128
NP = 10240
ROWS_PER_TILE = NP // NS
EP = NW * 80 * G
NPE = NW * 3 * G
EROWS = EP // G
GROUPS = EROWS // NW
NROWS = NPE // G


def _mm_body(x_ref, w_ref, o_ref):
    o_ref[...] = jnp.dot(x_ref[...], w_ref[...],
                         preferred_element_type=jnp.float32)


def _sc_body(xw, src2, dst2, nidx2, embed,
             accp, degp, embg,
             acc_sh, srcbuf, dstbuf, nbuf, rows_v, deg_l, sem):
    c = lax.axis_index("c")
    s = lax.axis_index("s")
    w = s * NC + c

    zeros16 = jnp.zeros((16,), jnp.float32)
    ones16 = jnp.ones((16,), jnp.float32)

    pltpu.sync_copy(src2.at[pl.ds(w * GROUPS, GROUPS)], srcbuf)
    pltpu.sync_copy(dst2.at[pl.ds(w * GROUPS, GROUPS)], dstbuf)
    pltpu.sync_copy(nidx2.at[pl.ds(w * 3, 3)], nbuf)

    def _zdeg(i, carry):
        deg_l[pl.ds(i * 16, 16)] = zeros16
        return carry
    lax.fori_loop(0, NP // 16, _zdeg, 0)

    def _zrow(i, carry):
        for j in range(D // 16):
            rows_v[i, pl.ds(j * 16, 16)] = zeros16
        return carry
    lax.fori_loop(0, G, _zrow, 0)
    for k in range(ROWS_PER_TILE // G):
        pltpu.sync_copy(rows_v, acc_sh.at[pl.ds(s * ROWS_PER_TILE + k * G, G)])

    plsc.subcore_barrier()

    def _edge(g, carry):
        pltpu.async_copy(xw.at[srcbuf.at[g]], rows_v, sem).wait()
        pltpu.sync_copy(rows_v, acc_sh.at[dstbuf.at[g]], add=True)
        for k in range(G // 16):
            dv = dstbuf[g, pl.ds(k * 16, 16)]
            plsc.addupdate_scatter(deg_l, [dv], ones16)
        return carry
    lax.fori_loop(0, GROUPS, _edge, 0)

    plsc.subcore_barrier()

    for t in range(3):
        pltpu.async_copy(embed.at[nbuf.at[t]], rows_v, sem).wait()
        pltpu.sync_copy(rows_v, embg.at[pl.ds(w * 3 * G + t * G, G)])

    pltpu.sync_copy(acc_sh.at[pl.ds(s * ROWS_PER_TILE, ROWS_PER_TILE)],
                    accp.at[pl.ds(c * NP + s * ROWS_PER_TILE, ROWS_PER_TILE)])
    pltpu.sync_copy(deg_l, degp.at[w])


def _comb_body(a0_ref, a1_ref, deg_ref, emb_ref, o_ref):
    h = a0_ref[...] + a1_ref[...]
    deg = jnp.sum(deg_ref[...], axis=0)
    h = h / jnp.maximum(deg, 1.0)[:, None]
    o_ref[:, 0, :] = emb_ref[...]
    o_ref[:, 1, :] = h


def kernel(x, edge_index, node_idx, W, embed):
    xw = pl.pallas_call(
        _mm_body,
        out_shape=jax.ShapeDtypeStruct((N, D), jnp.float32),
    )(x, W)

    src = edge_index[0]
    dst = edge_index[1]
    src2 = jnp.concatenate(
        [src, jnp.zeros((EP - E,), jnp.int32)]).reshape(EROWS, G)
    dst2 = jnp.concatenate(
        [dst, jnp.full((EP - E,), N, jnp.int32)]).reshape(EROWS, G)
    nidx2 = jnp.concatenate(
        [node_idx, jnp.zeros((NPE - N,), jnp.int32)]).reshape(NROWS, G)

    sc = pl.kernel(
        _sc_body,
        out_type=[
            jax.ShapeDtypeStruct((NC * NP, D), jnp.float32),
            jax.ShapeDtypeStruct((NW, NP), jnp.float32),
            jax.ShapeDtypeStruct((NPE, D), jnp.float32),
        ],
        mesh=plsc.VectorSubcoreMesh(core_axis_name="c", subcore_axis_name="s"),
        scratch_types=[
            pltpu.VMEM_SHARED((NP, D), jnp.float32),
            pltpu.VMEM((GROUPS, G), jnp.int32),
            pltpu.VMEM((GROUPS, G), jnp.int32),
            pltpu.VMEM((3, G), jnp.int32),
            pltpu.VMEM((G, D), jnp.float32),
            pltpu.VMEM((NP,), jnp.float32),
            pltpu.SemaphoreType.DMA,
        ],
    )
    accp, degp, embg = sc(xw, src2, dst2, nidx2, embed)

    BLK = 1024
    out = pl.pallas_call(
        _comb_body,
        grid=(NP // BLK,),
        in_specs=[
            pl.BlockSpec((BLK, D), lambda i: (i, 0)),
            pl.BlockSpec((BLK, D), lambda i: (NP // BLK + i, 0)),
            pl.BlockSpec((NW, BLK), lambda i: (0, i)),
            pl.BlockSpec((BLK, D), lambda i: (i, 0)),
        ],
        out_specs=pl.BlockSpec((BLK, 2, D), lambda i: (i, 0, 0)),
        out_shape=jax.ShapeDtypeStruct((NP, 2, D), jnp.float32),
    )(accp, accp, degp, embg)

    return out[:N]


# speedup vs baseline: 3.4206x; 3.4206x over previous
import functools

import jax
import jax.numpy as jnp
from jax import lax
from jax.experimental import pallas as pl
from jax.experimental.pallas import tpu as pltpu
from jax.experimental.pallas import tpu_sc as plsc

N = 10000
E = 320000
D = 128
NUM_EMB = 100000

NC = 2
NS = 16
NW = NC * NS

G = 128
NP = 10240
ROWS_PER_TILE = NP // NS
EP = NW * 80 * G
NPE = NW * 3 * G
EROWS = EP // G
GROUPS = EROWS // NW
NROWS = NPE // G


def _mm_body(x_ref, w_ref, o_ref):
    o_ref[...] = jnp.dot(x_ref[...], w_ref[...],
                         preferred_element_type=jnp.float32)


def _sc_body(xw, src2, dst2, nidx1, embed,
             accp, degp, embg,
             acc_sh, srcbuf, dstbuf, nbuf, rows_v, deg_l, sem):
    c = lax.axis_index("c")
    s = lax.axis_index("s")
    w = s * NC + c

    zeros16 = jnp.zeros((16,), jnp.float32)
    ones16 = jnp.ones((16,), jnp.float32)

    pltpu.sync_copy(src2.at[pl.ds(w * GROUPS, GROUPS)], srcbuf)
    pltpu.sync_copy(dst2.at[pl.ds(w * GROUPS, GROUPS)], dstbuf)
    pltpu.sync_copy(nidx1.at[pl.ds(w * (NPE // NW), NPE // NW)], nbuf)

    def _zdeg(i, carry):
        deg_l[pl.ds(i * 16, 16)] = zeros16
        return carry
    lax.fori_loop(0, NP // 16, _zdeg, 0)

    def _zrow(i, carry):
        for j in range(D // 16):
            rows_v[i, pl.ds(j * 16, 16)] = zeros16
        return carry
    lax.fori_loop(0, G, _zrow, 0)
    for k in range(ROWS_PER_TILE // G):
        pltpu.sync_copy(rows_v, acc_sh.at[pl.ds(s * ROWS_PER_TILE + k * G, G)])

    plsc.subcore_barrier()

    def _edge(g, carry):
        pltpu.async_copy(xw.at[srcbuf.at[g]], rows_v, sem).wait()
        pltpu.sync_copy(rows_v, acc_sh.at[dstbuf.at[g]], add=True)
        for k in range(G // 16):
            dv = dstbuf[g, pl.ds(k * 16, 16)]
            plsc.addupdate_scatter(deg_l, [dv], ones16)
        return carry
    lax.fori_loop(0, GROUPS, _edge, 0)

    plsc.subcore_barrier()

    for t in range(3):
        pltpu.async_copy(embed.at[nbuf.at[pl.ds(t * G, G)]], rows_v, sem).wait()
        pltpu.sync_copy(rows_v, embg.at[pl.ds(w * 3 * G + t * G, G)])

    pltpu.sync_copy(acc_sh.at[pl.ds(s * ROWS_PER_TILE, ROWS_PER_TILE)],
                    accp.at[pl.ds(c * NP + s * ROWS_PER_TILE, ROWS_PER_TILE)])
    pltpu.sync_copy(deg_l, degp.at[w])


def _comb_body(a0_ref, a1_ref, deg_ref, emb_ref, o_ref):
    h = a0_ref[...] + a1_ref[...]
    deg = jnp.sum(deg_ref[...], axis=0)
    h = h / jnp.maximum(deg, 1.0)[:, None]
    o_ref[:, 0, :] = emb_ref[...]
    o_ref[:, 1, :] = h


def kernel(x, edge_index, node_idx, W, embed):
    xw = pl.pallas_call(
        _mm_body,
        out_shape=jax.ShapeDtypeStruct((N, D), jnp.float32),
    )(x, W)

    src = edge_index[0]
    dst = edge_index[1]
    src2 = jnp.concatenate(
        [src, jnp.zeros((EP - E,), jnp.int32)]).reshape(EROWS, G)
    dst2 = jnp.concatenate(
        [dst, jnp.full((EP - E,), N, jnp.int32)]).reshape(EROWS, G)
    nidx1 = jnp.concatenate(
        [node_idx, jnp.zeros((NPE - N,), jnp.int32)])

    sc = pl.kernel(
        _sc_body,
        out_type=[
            jax.ShapeDtypeStruct((NC * NP, D), jnp.float32),
            jax.ShapeDtypeStruct((NW, NP), jnp.float32),
            jax.ShapeDtypeStruct((NPE, D), jnp.float32),
        ],
        mesh=plsc.VectorSubcoreMesh(core_axis_name="c", subcore_axis_name="s"),
        compiler_params=pltpu.CompilerParams(needs_layout_passes=False),
        scratch_types=[
            pltpu.VMEM_SHARED((NP, D), jnp.float32),
            pltpu.VMEM((GROUPS, G), jnp.int32),
            pltpu.VMEM((GROUPS, G), jnp.int32),
            pltpu.VMEM((NPE // NW,), jnp.int32),
            pltpu.VMEM((G, D), jnp.float32),
            pltpu.VMEM((NP,), jnp.float32),
            pltpu.SemaphoreType.DMA,
        ],
    )
    accp, degp, embg = sc(xw, src2, dst2, nidx1, embed)

    BLK = 1024
    out = pl.pallas_call(
        _comb_body,
        grid=(NP // BLK,),
        in_specs=[
            pl.BlockSpec((BLK, D), lambda i: (i, 0)),
            pl.BlockSpec((BLK, D), lambda i: (NP // BLK + i, 0)),
            pl.BlockSpec((NW, BLK), lambda i: (0, i)),
            pl.BlockSpec((BLK, D), lambda i: (i, 0)),
        ],
        out_specs=pl.BlockSpec((BLK, 2, D), lambda i: (i, 0, 0)),
        out_shape=jax.ShapeDtypeStruct((NP, 2, D), jnp.float32),
    )(accp, accp, degp, embg)

    return out[:N]


# speedup vs baseline: 3.7806x; 1.1052x over previous
import functools

import jax
import jax.numpy as jnp
from jax import lax
from jax.experimental import pallas as pl
from jax.experimental.pallas import tpu as pltpu
from jax.experimental.pallas import tpu_sc as plsc

N = 10000
E = 320000
D = 128
NUM_EMB = 100000

NC = 2
NS = 16
NW = NC * NS

G = 128
NP = 10240
ROWS_PER_TILE = NP // NS
GROUPS = 80
EP = NW * GROUPS * G
EROWS = EP // G
NPE = 10240
EMB_PER_TILE = NPE // NW
EMB_CHUNKS = ((0, 128), (128, 128), (256, 64))


def _mm_body(x_ref, w_ref, o_ref):
    o_ref[...] = jnp.dot(x_ref[...], w_ref[...],
                         preferred_element_type=jnp.float32)


def _sc_body(xw, src1, dst1, nidx1, embed,
             accp, degp, embg,
             acc_sh, sring, dring, nbuf, rows2, deg_l, semg, sems, semi):
    c = lax.axis_index("c")
    s = lax.axis_index("s")
    w = s * NC + c
    ebase = w * GROUPS * G

    zeros16 = jnp.zeros((16,), jnp.float32)
    ones16 = jnp.ones((16,), jnp.float32)

    pltpu.sync_copy(nidx1.at[pl.ds(w * EMB_PER_TILE, EMB_PER_TILE)], nbuf)

    def _zdeg(i, carry):
        deg_l[pl.ds(i * 16, 16)] = zeros16
        return carry
    lax.fori_loop(0, NP // 16, _zdeg, 0)

    def _zrow(i, carry):
        for j in range(D // 16):
            rows2[0, i, pl.ds(j * 16, 16)] = zeros16
        return carry
    lax.fori_loop(0, G, _zrow, 0)
    for k in range(ROWS_PER_TILE // G):
        pltpu.sync_copy(rows2.at[0],
                        acc_sh.at[pl.ds(s * ROWS_PER_TILE + k * G, G)])

    def idx_load(g, r):
        pltpu.async_copy(src1.at[pl.ds(ebase + g * G, G)], sring.at[r], semi)
        pltpu.async_copy(dst1.at[pl.ds(ebase + g * G, G)], dring.at[r], semi)

    def idx_wait(g, r):
        pltpu.make_async_copy(
            src1.at[pl.ds(ebase + g * G, G)], sring.at[r], semi).wait()
        pltpu.make_async_copy(
            dst1.at[pl.ds(ebase + g * G, G)], dring.at[r], semi).wait()

    idx_load(0, 0)
    idx_wait(0, 0)
    idx_load(1, 1)
    pltpu.async_copy(xw.at[sring.at[0]], rows2.at[0], semg)

    plsc.subcore_barrier()

    def _edge(g, carry):
        b = lax.rem(g, 2)
        r = lax.rem(g, 4)
        r1 = lax.rem(g + 1, 4)
        r2 = lax.rem(g + 2, 4)

        pltpu.make_async_copy(xw.at[sring.at[r]], rows2.at[b], semg).wait()

        @pl.when(g >= 1)
        def _():
            pltpu.make_async_copy(
                rows2.at[1 - b], acc_sh.at[dring.at[r]], sems).wait()

        @pl.when(g + 1 < GROUPS)
        def _():
            idx_wait(g + 1, r1)
            pltpu.async_copy(xw.at[sring.at[r1]], rows2.at[1 - b], semg)

        @pl.when(g + 2 < GROUPS)
        def _():
            idx_load(g + 2, r2)

        pltpu.async_copy(rows2.at[b], acc_sh.at[dring.at[r]], sems, add=True)

        for k in range(G // 16):
            dv = dring[r, pl.ds(k * 16, 16)]
            plsc.addupdate_scatter(deg_l, [dv], ones16)
        return carry

    lax.fori_loop(0, GROUPS, _edge, 0)
    pltpu.make_async_copy(
        rows2.at[1], acc_sh.at[dring.at[3]], sems).wait()

    plsc.subcore_barrier()

    for off, cnt in EMB_CHUNKS:
        pltpu.async_copy(embed.at[nbuf.at[pl.ds(off, cnt)]],
                         rows2.at[0, pl.ds(0, cnt)], semg).wait()
        pltpu.sync_copy(rows2.at[0, pl.ds(0, cnt)],
                        embg.at[pl.ds(w * EMB_PER_TILE + off, cnt)])

    pltpu.sync_copy(acc_sh.at[pl.ds(s * ROWS_PER_TILE, ROWS_PER_TILE)],
                    accp.at[c, pl.ds(s * ROWS_PER_TILE, ROWS_PER_TILE)])
    pltpu.sync_copy(deg_l, degp.at[w])


def _comb_body(a0_ref, a1_ref, deg_ref, emb_ref, o_ref):
    h = a0_ref[0] + a1_ref[0]
    deg = jnp.sum(deg_ref[...], axis=0)
    h = h / jnp.maximum(deg, 1.0)[:, None]
    o_ref[:, 0, :] = emb_ref[...]
    o_ref[:, 1, :] = h


def kernel(x, edge_index, node_idx, W, embed):
    xw = pl.pallas_call(
        _mm_body,
        out_shape=jax.ShapeDtypeStruct((N, D), jnp.float32),
    )(x, W)

    src = edge_index[0]
    dst = edge_index[1]
    src1 = jnp.concatenate([src, jnp.zeros((EP - E,), jnp.int32)])
    dst1 = jnp.concatenate([dst, jnp.full((EP - E,), N, jnp.int32)])
    nidx1 = jnp.concatenate(
        [node_idx, jnp.zeros((NPE - N,), jnp.int32)])

    sc = pl.kernel(
        _sc_body,
        out_type=[
            jax.ShapeDtypeStruct((NC, NP, D), jnp.float32),
            jax.ShapeDtypeStruct((NW, NP), jnp.float32),
            jax.ShapeDtypeStruct((NPE, D), jnp.float32),
        ],
        mesh=plsc.VectorSubcoreMesh(core_axis_name="c", subcore_axis_name="s"),
        compiler_params=pltpu.CompilerParams(needs_layout_passes=False),
        scratch_types=[
            pltpu.VMEM_SHARED((NP, D), jnp.float32),
            pltpu.VMEM((4, G), jnp.int32),
            pltpu.VMEM((4, G), jnp.int32),
            pltpu.VMEM((EMB_PER_TILE,), jnp.int32),
            pltpu.VMEM((2, G, D), jnp.float32),
            pltpu.VMEM((NP,), jnp.float32),
            pltpu.SemaphoreType.DMA,
            pltpu.SemaphoreType.DMA,
            pltpu.SemaphoreType.DMA,
        ],
    )
    accp, degp, embg = sc(xw, src1, dst1, nidx1, embed)

    BLK = 1024
    out = pl.pallas_call(
        _comb_body,
        grid=(NP // BLK,),
        in_specs=[
            pl.BlockSpec((1, BLK, D), lambda i: (0, i, 0)),
            pl.BlockSpec((1, BLK, D), lambda i: (1, i, 0)),
            pl.BlockSpec((NW, BLK), lambda i: (0, i)),
            pl.BlockSpec((BLK, D), lambda i: (i, 0)),
        ],
        out_specs=pl.BlockSpec((BLK, 2, D), lambda i: (i, 0, 0)),
        out_shape=jax.ShapeDtypeStruct((NP, 2, D), jnp.float32),
    )(accp, accp, degp, embg)

    return out[:N]


# speedup vs baseline: 3.9440x; 1.0432x over previous
import functools

import jax
import jax.numpy as jnp
from jax import lax
from jax.experimental import pallas as pl
from jax.experimental.pallas import tpu as pltpu
from jax.experimental.pallas import tpu_sc as plsc

N = 10000
E = 320000
D = 128
NUM_EMB = 100000

NC = 2
NS = 16
NW = NC * NS

G = 128
NP = 10240
ROWS_PER_TILE = NP // NS
GROUPS = 80
EP = NW * GROUPS * G
EROWS = EP // G
NPE = 10240
EMB_PER_TILE = NPE // NW
EMB_CHUNKS = ((0, 128), (128, 128), (256, 64))


def _mm_body(x_ref, w_ref, o_ref):
    o_ref[...] = jnp.dot(x_ref[...], w_ref[...],
                         preferred_element_type=jnp.float32)


def _sc_body(xw, src1, dst1, nidx1, embed,
             accp, degp, embg,
             acc_sh, sring, dring, nbuf, rows2, deg_l, semg, sems, semi):
    c = lax.axis_index("c")
    s = lax.axis_index("s")
    w = s * NC + c
    ebase = w * GROUPS * G

    zeros16 = jnp.zeros((16,), jnp.float32)
    ones16 = jnp.ones((16,), jnp.float32)

    pltpu.sync_copy(nidx1.at[pl.ds(w * EMB_PER_TILE, EMB_PER_TILE)], nbuf)

    def _zdeg(i, carry):
        deg_l[pl.ds(i * 16, 16)] = zeros16
        return carry
    lax.fori_loop(0, NP // 16, _zdeg, 0)

    def _zrow(i, carry):
        for j in range(D // 16):
            rows2[0, i, pl.ds(j * 16, 16)] = zeros16
        return carry
    lax.fori_loop(0, G, _zrow, 0)
    for k in range(ROWS_PER_TILE // G):
        pltpu.sync_copy(rows2.at[0],
                        acc_sh.at[pl.ds(s * ROWS_PER_TILE + k * G, G)])

    def idx_load(g, r):
        pltpu.async_copy(src1.at[pl.ds(ebase + g * G, G)], sring.at[r], semi)
        pltpu.async_copy(dst1.at[pl.ds(ebase + g * G, G)], dring.at[r], semi)

    def idx_wait(g, r):
        pltpu.make_async_copy(
            src1.at[pl.ds(ebase + g * G, G)], sring.at[r], semi).wait()
        pltpu.make_async_copy(
            dst1.at[pl.ds(ebase + g * G, G)], dring.at[r], semi).wait()

    idx_load(0, 0)
    idx_wait(0, 0)
    idx_load(1, 1)
    pltpu.async_copy(xw.at[sring.at[0]], rows2.at[0], semg)

    plsc.subcore_barrier()

    def _edge(g, carry):
        b = lax.rem(g, 2)
        r = lax.rem(g, 4)
        r1 = lax.rem(g + 1, 4)
        r2 = lax.rem(g + 2, 4)

        pltpu.make_async_copy(xw.at[sring.at[r]], rows2.at[b], semg).wait()

        @pl.when(g >= 1)
        def _():
            pltpu.make_async_copy(
                rows2.at[1 - b], acc_sh.at[dring.at[r]], sems).wait()

        @pl.when(g + 1 < GROUPS)
        def _():
            idx_wait(g + 1, r1)
            pltpu.async_copy(xw.at[sring.at[r1]], rows2.at[1 - b], semg)

        @pl.when(g + 2 < GROUPS)
        def _():
            idx_load(g + 2, r2)

        pltpu.async_copy(rows2.at[b], acc_sh.at[dring.at[r]], sems, add=True)

        for k in range(G // 16):
            dv = dring[r, pl.ds(k * 16, 16)]
            plsc.addupdate_scatter(deg_l, [dv], ones16)
        return carry

    lax.fori_loop(0, GROUPS, _edge, 0)
    pltpu.make_async_copy(
        rows2.at[1], acc_sh.at[dring.at[3]], sems).wait()

    plsc.subcore_barrier()

    for off, cnt in EMB_CHUNKS:
        pltpu.async_copy(embed.at[nbuf.at[pl.ds(off, cnt)]],
                         rows2.at[0, pl.ds(0, cnt)], semg).wait()
        pltpu.sync_copy(rows2.at[0, pl.ds(0, cnt)],
                        embg.at[pl.ds(w * EMB_PER_TILE + off, cnt)])

    pltpu.sync_copy(acc_sh.at[pl.ds(s * ROWS_PER_TILE, ROWS_PER_TILE)],
                    accp.at[c, pl.ds(s * ROWS_PER_TILE, ROWS_PER_TILE)])
    pltpu.sync_copy(deg_l, degp.at[w])


def _comb_body(a0_ref, a1_ref, deg_ref, emb_ref, o_ref):
    h = a0_ref[0] + a1_ref[0]
    deg = jnp.sum(deg_ref[...], axis=0)
    h = h / jnp.maximum(deg, 1.0)[:, None]
    o_ref[:, 0, :] = emb_ref[...]
    o_ref[:, 1, :] = h


def kernel(x, edge_index, node_idx, W, embed):
    xw2 = pl.pallas_call(
        _mm_body,
        grid=(NC,),
        in_specs=[
            pl.BlockSpec((N, D), lambda i: (0, 0)),
            pl.BlockSpec((D, D), lambda i: (0, 0)),
        ],
        out_specs=pl.BlockSpec((N, D), lambda i: (i, 0)),
        out_shape=jax.ShapeDtypeStruct((NC * N, D), jnp.float32),
    )(x, W)

    src = edge_index[0]
    dst = edge_index[1]
    core_of_edge = (jnp.arange(EP, dtype=jnp.int32) // (GROUPS * G)) % NC
    src1 = (jnp.concatenate([src, jnp.zeros((EP - E,), jnp.int32)])
            + core_of_edge * N)
    dst1 = jnp.concatenate([dst, jnp.full((EP - E,), N, jnp.int32)])
    nidx1 = jnp.concatenate(
        [node_idx, jnp.zeros((NPE - N,), jnp.int32)])

    sc = pl.kernel(
        _sc_body,
        out_type=[
            jax.ShapeDtypeStruct((NC, NP, D), jnp.float32),
            jax.ShapeDtypeStruct((NW, NP), jnp.float32),
            jax.ShapeDtypeStruct((NPE, D), jnp.float32),
        ],
        mesh=plsc.VectorSubcoreMesh(core_axis_name="c", subcore_axis_name="s"),
        compiler_params=pltpu.CompilerParams(needs_layout_passes=False),
        scratch_types=[
            pltpu.VMEM_SHARED((NP, D), jnp.float32),
            pltpu.VMEM((4, G), jnp.int32),
            pltpu.VMEM((4, G), jnp.int32),
            pltpu.VMEM((EMB_PER_TILE,), jnp.int32),
            pltpu.VMEM((2, G, D), jnp.float32),
            pltpu.VMEM((NP,), jnp.float32),
            pltpu.SemaphoreType.DMA,
            pltpu.SemaphoreType.DMA,
            pltpu.SemaphoreType.DMA,
        ],
    )
    accp, degp, embg = sc(xw2, src1, dst1, nidx1, embed)

    BLK = 1024
    out = pl.pallas_call(
        _comb_body,
        grid=(NP // BLK,),
        in_specs=[
            pl.BlockSpec((1, BLK, D), lambda i: (0, i, 0)),
            pl.BlockSpec((1, BLK, D), lambda i: (1, i, 0)),
            pl.BlockSpec((NW, BLK), lambda i: (0, i)),
            pl.BlockSpec((BLK, D), lambda i: (i, 0)),
        ],
        out_specs=pl.BlockSpec((BLK, 2, D), lambda i: (i, 0, 0)),
        out_shape=jax.ShapeDtypeStruct((NP, 2, D), jnp.float32),
    )(accp, accp, degp, embg)

    return out[:N]
